# baseline (speedup 1.0000x reference)
import functools
import math

import jax
import jax.numpy as jnp
from jax import lax
from jax.experimental import pallas as pl
from jax.experimental.pallas import tpu as pltpu
from jax.experimental.pallas import tpu_sc as plsc

MEMORY_DIM = 512
NUM_EPISODES = 10000
N_PAD = 10240
BATCH = 1024
BB = 128
K = 10
NBLK = N_PAD // 128

_NEG = -1e30


def _topk_body(q_ref, ep_ref, w_ref, b_ref, vals_ref, idx_ref):
    q = q_ref[...]
    w = w_ref[...]
    bias = b_ref[...]
    proj = lax.dot_general(q, w, (((1,), (1,)), ((), ())),
                           preferred_element_type=jnp.float32) + bias
    ep = ep_ref[...]
    scores = lax.dot_general(proj, ep, (((1,), (1,)), ((), ())),
                             preferred_element_type=jnp.float32)
    scores = scores * (1.0 / math.sqrt(MEMORY_DIM))
    col = lax.broadcasted_iota(jnp.int32, (BB, N_PAD), 1)
    scores = jnp.where(col < NUM_EPISODES, scores, _NEG)
    m = jnp.max(scores, axis=1, keepdims=True)
    denom = jnp.sum(jnp.exp(scores - m), axis=1, keepdims=True)
    bmax = jnp.max(scores.reshape(BB, NBLK, 128), axis=2)
    blockid = lax.shift_right_logical(col, 7)
    gcol = lax.broadcasted_iota(jnp.int32, (BB, NBLK), 1)
    sub = lax.broadcasted_iota(jnp.int32, (BB, 128), 1)
    work = bmax
    cands, colss = [], []
    for _ in range(K):
        vi = jnp.max(work, axis=1, keepdims=True)
        gi = jnp.min(jnp.where(work == vi, gcol, jnp.int32(NBLK)),
                     axis=1, keepdims=True)
        work = jnp.where(gcol == gi, _NEG, work)
        ext = jnp.where(blockid == gi, scores, 0.0)
        cands.append(jnp.sum(ext.reshape(BB, NBLK, 128), axis=1))
        colss.append(gi * 128 + sub)
    cv = jnp.concatenate(cands, axis=1)
    cols = jnp.concatenate(colss, axis=1)
    big = jnp.int32(1 << 30)
    vals, idxs = [], []
    for _ in range(K):
        vi = jnp.max(cv, axis=1, keepdims=True)
        ci = jnp.min(jnp.where(cv == vi, cols, big), axis=1, keepdims=True)
        vals.append(vi)
        idxs.append(ci)
        cv = jnp.where(cols == ci, _NEG, cv)
    v = jnp.concatenate(vals, axis=1)
    i = jnp.concatenate(idxs, axis=1)
    vals_ref[...] = jnp.exp(v - m) / denom
    idx_ref[...] = i


_topk_call = pl.pallas_call(
    _topk_body,
    grid=(BATCH // BB,),
    in_specs=[
        pl.BlockSpec((BB, MEMORY_DIM), lambda i: (i, 0)),
        pl.BlockSpec((N_PAD, MEMORY_DIM), lambda i: (0, 0)),
        pl.BlockSpec((MEMORY_DIM, MEMORY_DIM), lambda i: (0, 0)),
        pl.BlockSpec((1, MEMORY_DIM), lambda i: (0, 0)),
    ],
    out_specs=[
        pl.BlockSpec((BB, K), lambda i: (i, 0)),
        pl.BlockSpec((BB, K), lambda i: (i, 0)),
    ],
    out_shape=[
        jax.ShapeDtypeStruct((BATCH, K), jnp.float32),
        jax.ShapeDtypeStruct((BATCH, K), jnp.int32),
    ],
)


_NC, _NS = 2, 16
_NW = _NC * _NS
_BG = BATCH * K
_B_PER_W = _BG // _NW
_CH = 80
_NCHUNK = _B_PER_W // _CH


@functools.lru_cache(maxsize=1)
def _gather_call():
    mesh = plsc.VectorSubcoreMesh(core_axis_name="c", subcore_axis_name="s")

    @functools.partial(
        pl.kernel,
        mesh=mesh,
        compiler_params=pltpu.CompilerParams(needs_layout_passes=False),
        out_type=jax.ShapeDtypeStruct((_BG, MEMORY_DIM), jnp.float32),
        scratch_types=[
            pltpu.VMEM((_CH,), jnp.int32),
            pltpu.VMEM((_CH, MEMORY_DIM), jnp.float32),
            pltpu.SemaphoreType.DMA,
        ],
    )
    def _gather_rows(idx_hbm, table_hbm, out_hbm, idx_v, rows_v, sem):
        wid = lax.axis_index("s") * _NC + lax.axis_index("c")
        base = wid * _B_PER_W
        for c in range(_NCHUNK):
            off = base + c * _CH
            pltpu.sync_copy(idx_hbm.at[pl.ds(off, _CH)], idx_v)
            pltpu.async_copy(table_hbm.at[idx_v], rows_v, sem).wait()
            pltpu.sync_copy(rows_v, out_hbm.at[pl.ds(off, _CH)])

    return _gather_rows


def kernel(query, episodes, W, b, k):
    ep_pad = jnp.pad(episodes, ((0, N_PAD - NUM_EPISODES), (0, 0)))
    vals, idx = _topk_call(query, ep_pad, W, b.reshape(1, MEMORY_DIM))
    rows = _gather_call()(idx.reshape(-1), episodes)
    return rows.reshape(BATCH, K, MEMORY_DIM), vals


# speedup vs baseline: 1.5188x; 1.5188x over previous
import functools
import math

import jax
import jax.numpy as jnp
from jax import lax
from jax.experimental import pallas as pl
from jax.experimental.pallas import tpu as pltpu
from jax.experimental.pallas import tpu_sc as plsc

MEMORY_DIM = 512
NUM_EPISODES = 10000
N_PAD = 10240
BATCH = 1024
BB = 128
SB = 256
K = 10
NBLK = N_PAD // 128
NC_ROW = K * 128

_NEG = -1e30


def _score_body(q_ref, ep_ref, w_ref, b_ref, sc3_ref, grp_ref, md_ref):
    q = q_ref[...]
    w = w_ref[...]
    bias = b_ref[...]
    proj = lax.dot_general(q, w, (((1,), (1,)), ((), ())),
                           preferred_element_type=jnp.float32) + bias
    ep = ep_ref[...]
    scores = lax.dot_general(proj, ep, (((1,), (1,)), ((), ())),
                             preferred_element_type=jnp.float32)
    scores = scores * (1.0 / math.sqrt(MEMORY_DIM))
    col = lax.broadcasted_iota(jnp.int32, (BB, N_PAD), 1)
    scores = jnp.where(col < NUM_EPISODES, scores, _NEG)
    m = jnp.max(scores, axis=1, keepdims=True)
    denom = jnp.sum(jnp.exp(scores - m), axis=1, keepdims=True)
    md_ref[...] = jnp.concatenate([m, denom], axis=1)
    s3 = scores.reshape(BB, NBLK, 128)
    sc3_ref[...] = s3
    work = jnp.max(s3, axis=2)
    gcol = lax.broadcasted_iota(jnp.int32, (BB, NBLK), 1)
    gids = []
    for _ in range(K):
        vi = jnp.max(work, axis=1, keepdims=True)
        gi = jnp.min(jnp.where(work == vi, gcol, jnp.int32(NBLK)),
                     axis=1, keepdims=True)
        gids.append(gi)
        work = jnp.where(gcol == gi, _NEG, work)
    grp_ref[...] = jnp.concatenate(gids, axis=1)


_score_call = pl.pallas_call(
    _score_body,
    grid=(BATCH // BB,),
    in_specs=[
        pl.BlockSpec((BB, MEMORY_DIM), lambda i: (i, 0)),
        pl.BlockSpec((N_PAD, MEMORY_DIM), lambda i: (0, 0)),
        pl.BlockSpec((MEMORY_DIM, MEMORY_DIM), lambda i: (0, 0)),
        pl.BlockSpec((1, MEMORY_DIM), lambda i: (0, 0)),
    ],
    out_specs=[
        pl.BlockSpec((BB, NBLK, 128), lambda i: (i, 0, 0)),
        pl.BlockSpec((BB, K), lambda i: (i, 0)),
        pl.BlockSpec((BB, 2), lambda i: (i, 0)),
    ],
    out_shape=[
        jax.ShapeDtypeStruct((BATCH, NBLK, 128), jnp.float32),
        jax.ShapeDtypeStruct((BATCH, K), jnp.int32),
        jax.ShapeDtypeStruct((BATCH, 2), jnp.float32),
    ],
)


def _select_body(c_ref, grp_ref, md_ref, vals_ref, idx_ref):
    cands = c_ref[...].reshape(SB, NC_ROW)
    grp = grp_ref[...]
    sub = lax.broadcasted_iota(jnp.int32, (SB, K, 128), 2)
    cols = (grp[:, :, None] * 128 + sub).reshape(SB, NC_ROW)
    md = md_ref[...]
    big = jnp.int32(1 << 30)
    work = cands
    vals, idxs = [], []
    for _ in range(K):
        vi = jnp.max(work, axis=1, keepdims=True)
        ci = jnp.min(jnp.where(work == vi, cols, big), axis=1, keepdims=True)
        vals.append(vi)
        idxs.append(ci)
        work = jnp.where(cols == ci, _NEG, work)
    v = jnp.concatenate(vals, axis=1)
    i = jnp.concatenate(idxs, axis=1)
    m = md[:, 0:1]
    den = md[:, 1:2]
    vals_ref[...] = jnp.exp(v - m) / den
    idx_ref[...] = i


_select_call = pl.pallas_call(
    _select_body,
    grid=(BATCH // SB,),
    in_specs=[
        pl.BlockSpec((SB * K, 128), lambda i: (i, 0)),
        pl.BlockSpec((SB, K), lambda i: (i, 0)),
        pl.BlockSpec((SB, 2), lambda i: (i, 0)),
    ],
    out_specs=[
        pl.BlockSpec((SB, K), lambda i: (i, 0)),
        pl.BlockSpec((SB, K), lambda i: (i, 0)),
    ],
    out_shape=[
        jax.ShapeDtypeStruct((BATCH, K), jnp.float32),
        jax.ShapeDtypeStruct((BATCH, K), jnp.int32),
    ],
)


_NC, _NS = 2, 16
_NW = _NC * _NS
_BG = BATCH * K
RW = BATCH // _NW
NCAND = RW * K
_CH = 80


@functools.lru_cache(maxsize=1)
def _blk_call():
    mesh = plsc.VectorSubcoreMesh(core_axis_name="c", subcore_axis_name="s")

    @functools.partial(
        pl.kernel,
        mesh=mesh,
        compiler_params=pltpu.CompilerParams(needs_layout_passes=False),
        out_type=jax.ShapeDtypeStruct((_BG, 128), jnp.float32),
        scratch_types=[
            pltpu.VMEM((NCAND,), jnp.int32),
            pltpu.VMEM((NCAND,), jnp.int32),
            pltpu.VMEM((NCAND, 128), jnp.float32),
            pltpu.SemaphoreType.DMA,
            pltpu.SemaphoreType.DMA,
        ],
    )
    def _blk_body(grp_hbm, sctab_hbm, out_hbm, grpv, blkidx, blocks,
                  sem, sem2):
        wid = lax.axis_index("s") * _NC + lax.axis_index("c")
        r0 = wid * RW
        cbase = r0 * K
        pltpu.sync_copy(grp_hbm.at[pl.ds(cbase, NCAND)], grpv)
        lane = lax.iota(jnp.int32, 16)
        for c in range(NCAND // 16):
            g = grpv[pl.ds(c * 16, 16)]
            p = lane + c * 16
            r_loc = ((p.astype(jnp.float32) / jnp.float32(K))
                     .astype(jnp.int32))
            blkidx[pl.ds(c * 16, 16)] = (r0 + r_loc) * NBLK + g
        cps = [
            pltpu.async_copy(
                sctab_hbm.at[blkidx.at[pl.ds(off, n)]],
                blocks.at[pl.ds(off, n)], s)
            for off, n, s in ((0, 128, sem), (128, 128, sem2))
        ]
        cp3 = pltpu.async_copy(
            sctab_hbm.at[blkidx.at[pl.ds(256, 64)]],
            blocks.at[pl.ds(256, 64)], sem)
        for cp in cps:
            cp.wait()
        cp3.wait()
        pltpu.sync_copy(blocks, out_hbm.at[pl.ds(cbase, NCAND)])

    return _blk_body


@functools.lru_cache(maxsize=1)
def _gather_call():
    mesh = plsc.VectorSubcoreMesh(core_axis_name="c", subcore_axis_name="s")

    @functools.partial(
        pl.kernel,
        mesh=mesh,
        compiler_params=pltpu.CompilerParams(needs_layout_passes=False),
        out_type=jax.ShapeDtypeStruct((_BG, MEMORY_DIM), jnp.float32),
        scratch_types=[
            pltpu.VMEM((_CH,), jnp.int32),
            pltpu.VMEM((_CH, MEMORY_DIM), jnp.float32),
            pltpu.SemaphoreType.DMA,
        ],
    )
    def _gather_rows(idx_hbm, table_hbm, out_hbm, idx_v, rows_v, sem):
        wid = lax.axis_index("s") * _NC + lax.axis_index("c")
        base = wid * NCAND
        for c in range(NCAND // _CH):
            off = base + c * _CH
            pltpu.sync_copy(idx_hbm.at[pl.ds(off, _CH)], idx_v)
            pltpu.async_copy(table_hbm.at[idx_v], rows_v, sem).wait()
            pltpu.sync_copy(rows_v, out_hbm.at[pl.ds(off, _CH)])

    return _gather_rows


def kernel(query, episodes, W, b, k):
    ep_pad = jnp.pad(episodes, ((0, N_PAD - NUM_EPISODES), (0, 0)))
    sc3, grp, md = _score_call(query, ep_pad, W, b.reshape(1, MEMORY_DIM))
    sctab = sc3.reshape(BATCH * NBLK, 128)
    cands = _blk_call()(grp.reshape(-1), sctab)
    vals, idx = _select_call(cands, grp, md)
    rows = _gather_call()(idx.reshape(-1), episodes)
    return rows.reshape(BATCH, K, MEMORY_DIM), vals


# speedup vs baseline: 1.5331x; 1.0094x over previous
import functools
import math

import jax
import jax.numpy as jnp
from jax import lax
from jax.experimental import pallas as pl
from jax.experimental.pallas import tpu as pltpu
from jax.experimental.pallas import tpu_sc as plsc

MEMORY_DIM = 512
NUM_EPISODES = 10000
N_PAD = 10240
BATCH = 1024
BB = 128
SB = 512
K = 10
NBLK = N_PAD // 128
NC_ROW = K * 128

_NEG = -1e30


def _score_body(q_ref, ep_ref, w_ref, b_ref, sc3_ref, grp_ref, md_ref):
    q = q_ref[...]
    w = w_ref[...]
    bias = b_ref[...]
    proj = lax.dot_general(q, w, (((1,), (1,)), ((), ())),
                           preferred_element_type=jnp.float32) + bias
    ep = ep_ref[...]
    scores = lax.dot_general(proj, ep, (((1,), (1,)), ((), ())),
                             preferred_element_type=jnp.float32)
    scores = scores * (1.0 / math.sqrt(MEMORY_DIM))
    col = lax.broadcasted_iota(jnp.int32, (BB, N_PAD), 1)
    scores = jnp.where(col < NUM_EPISODES, scores, _NEG)
    m = jnp.max(scores, axis=1, keepdims=True)
    denom = jnp.sum(jnp.exp(scores - m), axis=1, keepdims=True)
    md_ref[...] = jnp.concatenate([m, denom], axis=1)
    s3 = scores.reshape(BB, NBLK, 128)
    sc3_ref[...] = s3
    work = jnp.max(s3, axis=2)
    gcol = lax.broadcasted_iota(jnp.int32, (BB, NBLK), 1)
    gids = []
    for _ in range(K):
        vi = jnp.max(work, axis=1, keepdims=True)
        gi = jnp.min(jnp.where(work == vi, gcol, jnp.int32(NBLK)),
                     axis=1, keepdims=True)
        gids.append(gi)
        work = jnp.where(gcol == gi, _NEG, work)
    grp_ref[...] = jnp.concatenate(gids, axis=1)


_score_call = pl.pallas_call(
    _score_body,
    grid=(BATCH // BB,),
    in_specs=[
        pl.BlockSpec((BB, MEMORY_DIM), lambda i: (i, 0)),
        pl.BlockSpec((N_PAD, MEMORY_DIM), lambda i: (0, 0)),
        pl.BlockSpec((MEMORY_DIM, MEMORY_DIM), lambda i: (0, 0)),
        pl.BlockSpec((1, MEMORY_DIM), lambda i: (0, 0)),
    ],
    out_specs=[
        pl.BlockSpec((BB, NBLK, 128), lambda i: (i, 0, 0)),
        pl.BlockSpec((BB, K), lambda i: (i, 0)),
        pl.BlockSpec((BB, 2), lambda i: (i, 0)),
    ],
    out_shape=[
        jax.ShapeDtypeStruct((BATCH, NBLK, 128), jnp.float32),
        jax.ShapeDtypeStruct((BATCH, K), jnp.int32),
        jax.ShapeDtypeStruct((BATCH, 2), jnp.float32),
    ],
)


def _select_body(c_ref, grp_ref, md_ref, vals_ref, idx_ref):
    cands = c_ref[...].reshape(SB, NC_ROW)
    grp = grp_ref[...]
    sub = lax.broadcasted_iota(jnp.int32, (SB, K, 128), 2)
    cols = (grp[:, :, None] * 128 + sub).reshape(SB, NC_ROW)
    md = md_ref[...]
    big = jnp.int32(1 << 30)
    work = cands
    vals, idxs = [], []
    for _ in range(K):
        vi = jnp.max(work, axis=1, keepdims=True)
        ci = jnp.min(jnp.where(work == vi, cols, big), axis=1, keepdims=True)
        vals.append(vi)
        idxs.append(ci)
        work = jnp.where(cols == ci, _NEG, work)
    v = jnp.concatenate(vals, axis=1)
    i = jnp.concatenate(idxs, axis=1)
    m = md[:, 0:1]
    den = md[:, 1:2]
    vals_ref[...] = jnp.exp(v - m) / den
    idx_ref[...] = i


_select_call = pl.pallas_call(
    _select_body,
    grid=(BATCH // SB,),
    in_specs=[
        pl.BlockSpec((SB * K, 128), lambda i: (i, 0)),
        pl.BlockSpec((SB, K), lambda i: (i, 0)),
        pl.BlockSpec((SB, 2), lambda i: (i, 0)),
    ],
    out_specs=[
        pl.BlockSpec((SB, K), lambda i: (i, 0)),
        pl.BlockSpec((SB, K), lambda i: (i, 0)),
    ],
    out_shape=[
        jax.ShapeDtypeStruct((BATCH, K), jnp.float32),
        jax.ShapeDtypeStruct((BATCH, K), jnp.int32),
    ],
)


_NC, _NS = 2, 16
_NW = _NC * _NS
_BG = BATCH * K
RW = BATCH // _NW
NCAND = RW * K
_CH = 80


@functools.lru_cache(maxsize=1)
def _blk_call():
    mesh = plsc.VectorSubcoreMesh(core_axis_name="c", subcore_axis_name="s")

    @functools.partial(
        pl.kernel,
        mesh=mesh,
        compiler_params=pltpu.CompilerParams(needs_layout_passes=False),
        out_type=jax.ShapeDtypeStruct((_BG, 128), jnp.float32),
        scratch_types=[
            pltpu.VMEM((NCAND,), jnp.int32),
            pltpu.VMEM((NCAND,), jnp.int32),
            pltpu.VMEM((NCAND, 128), jnp.float32),
            pltpu.SemaphoreType.DMA,
            pltpu.SemaphoreType.DMA,
        ],
    )
    def _blk_body(grp_hbm, sctab_hbm, out_hbm, grpv, blkidx, blocks,
                  sem, sem2):
        wid = lax.axis_index("s") * _NC + lax.axis_index("c")
        r0 = wid * RW
        cbase = r0 * K
        pltpu.sync_copy(grp_hbm.at[pl.ds(cbase, NCAND)], grpv)
        lane = lax.iota(jnp.int32, 16)
        for c in range(NCAND // 16):
            g = grpv[pl.ds(c * 16, 16)]
            p = lane + c * 16
            r_loc = ((p.astype(jnp.float32) / jnp.float32(K))
                     .astype(jnp.int32))
            blkidx[pl.ds(c * 16, 16)] = (r0 + r_loc) * NBLK + g
        cps = [
            pltpu.async_copy(
                sctab_hbm.at[blkidx.at[pl.ds(off, n)]],
                blocks.at[pl.ds(off, n)], s)
            for off, n, s in ((0, 128, sem), (128, 128, sem2))
        ]
        cp3 = pltpu.async_copy(
            sctab_hbm.at[blkidx.at[pl.ds(256, 64)]],
            blocks.at[pl.ds(256, 64)], sem)
        for cp in cps:
            cp.wait()
        cp3.wait()
        pltpu.sync_copy(blocks, out_hbm.at[pl.ds(cbase, NCAND)])

    return _blk_body


@functools.lru_cache(maxsize=1)
def _gather_call():
    mesh = plsc.VectorSubcoreMesh(core_axis_name="c", subcore_axis_name="s")

    @functools.partial(
        pl.kernel,
        mesh=mesh,
        compiler_params=pltpu.CompilerParams(needs_layout_passes=False),
        out_type=jax.ShapeDtypeStruct((_BG, MEMORY_DIM), jnp.float32),
        scratch_types=[
            pltpu.VMEM((_CH,), jnp.int32),
            pltpu.VMEM((_CH, MEMORY_DIM), jnp.float32),
            pltpu.SemaphoreType.DMA,
        ],
    )
    def _gather_rows(idx_hbm, table_hbm, out_hbm, idx_v, rows_v, sem):
        wid = lax.axis_index("s") * _NC + lax.axis_index("c")
        base = wid * NCAND
        for c in range(NCAND // _CH):
            off = base + c * _CH
            pltpu.sync_copy(idx_hbm.at[pl.ds(off, _CH)], idx_v)
            pltpu.async_copy(table_hbm.at[idx_v], rows_v, sem).wait()
            pltpu.sync_copy(rows_v, out_hbm.at[pl.ds(off, _CH)])

    return _gather_rows


def kernel(query, episodes, W, b, k):
    ep_pad = jnp.pad(episodes, ((0, N_PAD - NUM_EPISODES), (0, 0)))
    sc3, grp, md = _score_call(query, ep_pad, W, b.reshape(1, MEMORY_DIM))
    sctab = sc3.reshape(BATCH * NBLK, 128)
    cands = _blk_call()(grp.reshape(-1), sctab)
    vals, idx = _select_call(cands, grp, md)
    rows = _gather_call()(idx.reshape(-1), episodes)
    return rows.reshape(BATCH, K, MEMORY_DIM), vals
